# baseline (speedup 1.0000x reference)
import functools

import jax
import jax.numpy as jnp
from jax import lax
from jax.experimental import pallas as pl
from jax.experimental.pallas import tpu as pltpu
from jax.experimental.pallas import tpu_sc as plsc

_NC, _NS = 2, 16
_NW = _NC * _NS
_TBS = 64
_HP = 24


def _mp_body(e_ref, w_ref, b_ref, o1_ref, o2_ref):
    acc = (
        jnp.dot(e_ref[...].astype(jnp.bfloat16), w_ref[...],
                preferred_element_type=jnp.float32)
        + b_ref[...]
    )
    w1 = o1_ref.shape[1]
    o1_ref[...] = acc[:, :w1]
    o2_ref[...] = acc[:, w1:]


def _make_m(emb_p, wt_p, b2, vocab):
    w1 = (vocab // 128) * 128
    return pl.pallas_call(
        _mp_body,
        out_shape=[
            jax.ShapeDtypeStruct((vocab, w1), jnp.float32),
            jax.ShapeDtypeStruct((vocab, vocab - w1), jnp.float32),
        ],
    )(emb_p, wt_p, b2)


def _tail_body(x_ref, mt_ref, o_ref):
    rows = x_ref.shape[0]
    vocab = mt_ref.shape[0]
    onehot = (
        x_ref[...] == lax.broadcasted_iota(jnp.int32, (rows, vocab), 1)
    ).astype(jnp.bfloat16)
    acc = jnp.dot(onehot, mt_ref[...].astype(jnp.bfloat16),
                  preferred_element_type=jnp.float32)
    o_ref[...] = acc.reshape(o_ref.shape)


def _make_tail(x2, mt, batch, hist):
    vocab, tw = mt.shape
    return pl.pallas_call(
        _tail_body,
        grid=(batch // _TBS,),
        in_specs=[
            pl.BlockSpec((_TBS * hist, 1), lambda i: (i, 0)),
            pl.BlockSpec((vocab, tw), lambda i: (0, 0)),
        ],
        out_specs=pl.BlockSpec((_TBS, hist, tw), lambda i: (i, 0, 0)),
        out_shape=jax.ShapeDtypeStruct((batch, hist, tw), jnp.float32),
    )(x2, mt)


def _sc_lookup(m1, ot, xp, batch, hist, vocab):
    spw = batch // _NW
    w1 = m1.shape[1]
    tw = vocab - w1
    mesh = plsc.VectorSubcoreMesh(core_axis_name="c", subcore_axis_name="s")

    @functools.partial(
        pl.kernel,
        mesh=mesh,
        out_type=jax.ShapeDtypeStruct((batch, hist, vocab), jnp.float32),
        scratch_types=[
            pltpu.VMEM((spw * _HP,), jnp.int32),
            pltpu.VMEM((1, hist, w1), jnp.float32),
            pltpu.VMEM((1, hist, w1), jnp.float32),
            pltpu.VMEM((8, w1), jnp.float32),
            pltpu.VMEM((8, w1), jnp.float32),
            pltpu.VMEM((1, hist, tw), jnp.float32),
            pltpu.VMEM((1, hist, tw), jnp.float32),
            pltpu.SemaphoreType.DMA,
            pltpu.SemaphoreType.DMA,
            pltpu.SemaphoreType.DMA,
            pltpu.SemaphoreType.DMA,
        ],
    )
    def k(m1_hbm, ot_hbm, x_hbm, out_hbm, idx_v, b3a, b3b, bxa, bxb,
          bta, btb, sg0, sg1, sw0, sw1):
        wid = lax.axis_index("s") * _NC + lax.axis_index("c")
        slab0 = wid * spw
        pltpu.sync_copy(x_hbm.at[pl.ds(slab0 * _HP, spw * _HP)], idx_v)
        b3 = (b3a, b3b)
        bx = (bxa, bxb)
        bt = (bta, btb)
        sg = (sg0, sg1)
        sw = (sw0, sw1)
        nmv = w1 // 16

        def in_descs(g, p):
            return [
                pltpu.make_async_copy(
                    m1_hbm.at[idx_v.at[pl.ds(g * _HP, 16)]],
                    b3[p].at[0, pl.ds(0, 16)], sg[p]),
                pltpu.make_async_copy(
                    m1_hbm.at[idx_v.at[pl.ds(g * _HP + 16, 8)]],
                    bx[p], sg[p]),
                pltpu.make_async_copy(
                    ot_hbm.at[pl.ds(slab0 + g, 1)], bt[p], sg[p]),
            ]

        def fixup(p):
            for r in range(hist - 16):
                for kk in range(nmv):
                    b3[p][0, 16 + r, pl.ds(kk * 16, 16)] = (
                        bx[p][r, pl.ds(kk * 16, 16)]
                    )

        def out_descs(g, p):
            rows = out_hbm.at[pl.ds(slab0 + g, 1)]
            return [
                pltpu.make_async_copy(
                    b3[p], rows.at[:, :, pl.ds(0, w1)], sw[p]),
                pltpu.make_async_copy(
                    bt[p], rows.at[:, :, pl.ds(w1, tw)], sw[p]),
            ]

        for p in (0, 1):
            for dsc in in_descs(p, p):
                dsc.start()

        def pair(h, carry):
            for p in (0, 1):
                g = 2 * h + p
                for dsc in in_descs(g, p):
                    dsc.wait()
                fixup(p)
                for dsc in out_descs(g, p):
                    dsc.start()

                @pl.when(g + 2 < spw)
                def _():
                    for dsc in out_descs(g, p):
                        dsc.wait()
                    for dsc in in_descs(g + 2, p):
                        dsc.start()

            return carry

        lax.fori_loop(0, spw // 2, pair, 0)
        for dsc in out_descs(spw - 2, 0):
            dsc.wait()
        for dsc in out_descs(spw - 1, 1):
            dsc.wait()

    return k(m1, ot, xp)


def kernel(x, emb_table, W_head, b_head):
    b, l = x.shape
    v, d = emb_table.shape
    xi = x.astype(jnp.int32)
    emb_p = jnp.pad(emb_table, ((0, 0), (0, 128 - d)))
    wt_p = jnp.pad(W_head.T, ((0, 128 - d), (0, 0))).astype(jnp.bfloat16)
    m1, mt = _make_m(emb_p, wt_p, b_head.reshape(1, v), v)
    ot = _make_tail(xi.reshape(b * l, 1), mt, b, l)
    xp = jnp.pad(xi, ((0, 0), (0, _HP - l)), mode="edge").reshape(b * _HP)
    return _sc_lookup(m1, ot, xp, b, l, v)


# speedup vs baseline: 1.7528x; 1.7528x over previous
import functools

import jax
import jax.numpy as jnp
from jax import lax
from jax.experimental import layout as jlayout
from jax.experimental import pallas as pl
from jax.experimental.pallas import tpu as pltpu
from jax.experimental.pallas import tpu_sc as plsc

_NC, _NS = 2, 16
_NW = _NC * _NS
_TBS = 64
_HP = 24


def _mp_body(e_ref, w_ref, b_ref, o1_ref, o2_ref):
    acc = (
        jnp.dot(e_ref[...].astype(jnp.bfloat16), w_ref[...],
                preferred_element_type=jnp.float32)
        + b_ref[...]
    )
    w1 = o1_ref.shape[1]
    o1_ref[...] = acc[:, :w1]
    o2_ref[...] = acc[:, w1:]


def _make_m(emb_p, wt_p, b2, vocab):
    w1 = (vocab // 128) * 128
    return pl.pallas_call(
        _mp_body,
        out_shape=[
            jax.ShapeDtypeStruct((vocab, w1), jnp.float32),
            jax.ShapeDtypeStruct((vocab, vocab - w1), jnp.float32),
        ],
    )(emb_p, wt_p, b2)


def _tail_body(x_ref, mt_ref, o_ref):
    rows = x_ref.shape[0]
    vocab = mt_ref.shape[0]
    onehot = (
        x_ref[...] == lax.broadcasted_iota(jnp.int32, (rows, vocab), 1)
    ).astype(jnp.bfloat16)
    acc = jnp.dot(onehot, mt_ref[...].astype(jnp.bfloat16),
                  preferred_element_type=jnp.float32)
    o_ref[...] = acc.reshape(o_ref.shape)


def _make_tail(x2, mt, batch, hist):
    vocab, tw = mt.shape
    return pl.pallas_call(
        _tail_body,
        grid=(batch // _TBS,),
        in_specs=[
            pl.BlockSpec((_TBS * hist, 1), lambda i: (i, 0)),
            pl.BlockSpec((vocab, tw), lambda i: (0, 0)),
        ],
        out_specs=pl.BlockSpec((_TBS, hist, tw), lambda i: (i, 0, 0)),
        out_shape=jax.ShapeDtypeStruct((batch, hist, tw), jnp.float32),
    )(x2, mt)


def _sc_lookup(m1, ot, xp, batch, hist, vocab):
    spw = batch // _NW
    w1 = m1.shape[1]
    tw = vocab - w1
    mesh = plsc.VectorSubcoreMesh(core_axis_name="c", subcore_axis_name="s")

    @functools.partial(
        pl.kernel,
        mesh=mesh,
        out_type=jax.ShapeDtypeStruct((batch, hist, vocab), jnp.float32),
        scratch_types=[
            pltpu.VMEM((spw * _HP,), jnp.int32),
            pltpu.VMEM((1, hist, w1), jnp.float32),
            pltpu.VMEM((1, hist, w1), jnp.float32),
            pltpu.VMEM((8, w1), jnp.float32),
            pltpu.VMEM((8, w1), jnp.float32),
            pltpu.VMEM((1, hist, tw), jnp.float32),
            pltpu.VMEM((1, hist, tw), jnp.float32),
            pltpu.SemaphoreType.DMA,
            pltpu.SemaphoreType.DMA,
            pltpu.SemaphoreType.DMA,
            pltpu.SemaphoreType.DMA,
        ],
    )
    def k(m1_hbm, ot_hbm, x_hbm, out_hbm, idx_v, b3a, b3b, bxa, bxb,
          bta, btb, sg0, sg1, sw0, sw1):
        wid = lax.axis_index("s") * _NC + lax.axis_index("c")
        slab0 = wid * spw
        pltpu.sync_copy(x_hbm.at[pl.ds(slab0 * _HP, spw * _HP)], idx_v)
        b3 = (b3a, b3b)
        bx = (bxa, bxb)
        bt = (bta, btb)
        sg = (sg0, sg1)
        sw = (sw0, sw1)
        nmv = w1 // 16

        def in_descs(g, p):
            return [
                pltpu.make_async_copy(
                    m1_hbm.at[idx_v.at[pl.ds(g * _HP, 16)]],
                    b3[p].at[0, pl.ds(0, 16)], sg[p]),
                pltpu.make_async_copy(
                    m1_hbm.at[idx_v.at[pl.ds(g * _HP + 16, 8)]],
                    bx[p], sg[p]),
                pltpu.make_async_copy(
                    ot_hbm.at[pl.ds(slab0 + g, 1)], bt[p], sg[p]),
            ]

        def fixup(p):
            for r in range(hist - 16):
                for kk in range(nmv):
                    b3[p][0, 16 + r, pl.ds(kk * 16, 16)] = (
                        bx[p][r, pl.ds(kk * 16, 16)]
                    )

        def out_descs(g, p):
            rows = out_hbm.at[pl.ds(slab0 + g, 1)]
            return [
                pltpu.make_async_copy(
                    b3[p], rows.at[:, :, pl.ds(0, w1)], sw[p]),
                pltpu.make_async_copy(
                    bt[p], rows.at[:, :, pl.ds(w1, tw)], sw[p]),
            ]

        for p in (0, 1):
            for dsc in in_descs(p, p):
                dsc.start()

        def pair(h, carry):
            for p in (0, 1):
                g = 2 * h + p
                for dsc in in_descs(g, p):
                    dsc.wait()
                fixup(p)
                for dsc in out_descs(g, p):
                    dsc.start()

                @pl.when(g + 2 < spw)
                def _():
                    for dsc in out_descs(g, p):
                        dsc.wait()
                    for dsc in in_descs(g + 2, p):
                        dsc.start()

            return carry

        lax.fori_loop(0, spw // 2, pair, 0)
        for dsc in out_descs(spw - 2, 0):
            dsc.wait()
        for dsc in out_descs(spw - 1, 1):
            dsc.wait()

    return k(m1, ot, xp)


def kernel(x, emb_table, W_head, b_head):
    b, l = x.shape
    v, d = emb_table.shape
    xi = x.astype(jnp.int32)
    emb_p = jnp.pad(emb_table, ((0, 0), (0, 128 - d)))
    wt_p = jnp.pad(W_head.T, ((0, 128 - d), (0, 0))).astype(jnp.bfloat16)
    m1, mt = _make_m(emb_p, wt_p, b_head.reshape(1, v), v)
    ot = _make_tail(xi.reshape(b * l, 1), mt, b, l)
    xp = jnp.pad(xi, ((0, 0), (0, _HP - l)), mode="edge").reshape(b * _HP)
    out = _sc_lookup(m1, ot, xp, b, l, v)
    return jlayout.with_layout_constraint(
        out, jlayout.Layout(major_to_minor=(0, 1, 2))
    )
